# baseline (speedup 1.0000x reference)
import functools

import jax
import jax.numpy as jnp
from jax import lax
from jax.experimental import pallas as pl
from jax.experimental.pallas import tpu as pltpu
from jax.experimental.pallas import tpu_sc as plsc

_N = 1024
_L = 16
_NC = 2
_NS = 16


def _auction_body(eps_ref, iters_ref, x1_ref, x2t_ref, ainv_ref, c_ref):
    n = _N
    x1 = x1_ref[0]
    x2t = x2t_ref[0]
    eps = eps_ref[0]
    iters = iters_ref[0]

    d0 = x1[:, 0:1] - x2t[0:1, :]
    d1 = x1[:, 1:2] - x2t[1:2, :]
    d2 = x1[:, 2:3] - x2t[2:3, :]
    c_ref[...] = (d0 * d0 + d1 * d1) + d2 * d2

    col = jax.lax.broadcasted_iota(jnp.int32, (1, n), 1)
    row = jax.lax.broadcasted_iota(jnp.int32, (n, 1), 0)
    neg_inf = jnp.float32(-jnp.inf)

    def round_body(_, carry):
        price, ass_inv, unass_i = carry
        unass = unass_i > 0
        pneg = -price
        vb = pneg - c_ref[...]
        best = jnp.max(vb, axis=1, keepdims=True)
        iseq = vb == best
        cnt = jnp.sum(iseq.astype(jnp.int32), axis=1, keepdims=True)
        bidx = jnp.min(jnp.where(iseq, col, n), axis=1, keepdims=True)
        m2 = jnp.max(jnp.where(iseq, neg_inf, vb), axis=1, keepdims=True)
        second = jnp.where(cnt > 1, best, m2)
        binc = best - second + eps
        bm = (bidx == col) & unass
        bb = jnp.where(bm, binc, neg_inf)
        maxinc = jnp.max(bb, axis=0, keepdims=True)
        winner = jnp.min(jnp.where(bb == maxinc, row, n), axis=0,
                         keepdims=True)
        has_bid = maxinc > neg_inf
        price2 = jnp.where(has_bid, price + maxinc, price)
        ass_inv2 = jnp.where(has_bid, winner, ass_inv)
        owned = jnp.any(ass_inv2 == row, axis=1, keepdims=True)
        return price2, ass_inv2, 1 - owned.astype(jnp.int32)

    price0 = jnp.zeros((1, n), jnp.float32)
    ass_inv0 = jnp.full((1, n), -1, jnp.int32)
    unass0 = jnp.ones((n, 1), jnp.int32)
    _, ass_inv, _ = jax.lax.fori_loop(
        0, iters, round_body, (price0, ass_inv0, unass0))
    ainv_ref[0] = ass_inv


def _epilogue_body(ainv_hbm, x1a_hbm, x1b_hbm, x1c_hbm, x2a_hbm, x2b_hbm,
                   x2c_hbm, ass_hbm, dist_hbm,
                   ainv_v, x1p0, x1p1, x1p2, x2p0, x2p1, x2p2, ass_v, dist_v):
    n = _N
    nb = ainv_hbm.shape[0]
    wid = lax.axis_index("s") * _NC + lax.axis_index("c")
    lane = jax.lax.broadcasted_iota(jnp.int32, (_L,), 0)

    def one_batch(b):
        pltpu.sync_copy(ainv_hbm.at[b], ainv_v)
        pltpu.sync_copy(x1a_hbm.at[b], x1p0)
        pltpu.sync_copy(x1b_hbm.at[b], x1p1)
        pltpu.sync_copy(x1c_hbm.at[b], x1p2)
        pltpu.sync_copy(x2a_hbm.at[b], x2p0)
        pltpu.sync_copy(x2b_hbm.at[b], x2p1)
        pltpu.sync_copy(x2c_hbm.at[b], x2p2)

        def chunk(ch, _c):
            sl = pl.ds(ch * _L, _L)
            jids = ch * _L + lane
            owners = ainv_v[sl]
            m = owners >= 0
            oc = jnp.maximum(owners, 0)
            d0 = plsc.load_gather(x1p0, [oc]) - x2p0[sl]
            d1 = plsc.load_gather(x1p1, [oc]) - x2p1[sl]
            d2 = plsc.load_gather(x1p2, [oc]) - x2p2[sl]
            dd = (d0 * d0 + d1 * d1) + d2 * d2
            plsc.store_scatter(dist_v, [oc], dd, mask=m)
            plsc.store_scatter(ass_v, [oc], jids, mask=m)
            return _c

        def init(ch, _c):
            sl = pl.ds(ch * _L, _L)
            ass_v[sl] = jnp.full((_L,), -1, jnp.int32)
            dist_v[sl] = jnp.zeros((_L,), jnp.float32)
            return _c

        lax.fori_loop(0, n // _L, init, 0)
        lax.fori_loop(0, n // _L, chunk, 0)
        pltpu.sync_copy(ass_v, ass_hbm.at[b])
        pltpu.sync_copy(dist_v, dist_hbm.at[b])

    @pl.when(wid < nb)
    def _():
        one_batch(wid)


def _epilogue(ainv, x1planes, x2planes):
    b = ainv.shape[0]
    mesh = plsc.VectorSubcoreMesh(
        core_axis_name="c", subcore_axis_name="s",
        num_cores=_NC, num_subcores=_NS)
    f = functools.partial(
        pl.kernel,
        out_type=[
            jax.ShapeDtypeStruct((b, _N), jnp.int32),
            jax.ShapeDtypeStruct((b, _N), jnp.float32),
        ],
        mesh=mesh,
        compiler_params=pltpu.CompilerParams(
            use_tc_tiling_on_sc=False, needs_layout_passes=False),
        scratch_types=[
            pltpu.VMEM((_N,), jnp.int32),
            pltpu.VMEM((_N,), jnp.float32),
            pltpu.VMEM((_N,), jnp.float32),
            pltpu.VMEM((_N,), jnp.float32),
            pltpu.VMEM((_N,), jnp.float32),
            pltpu.VMEM((_N,), jnp.float32),
            pltpu.VMEM((_N,), jnp.float32),
            pltpu.VMEM((_N,), jnp.int32),
            pltpu.VMEM((_N,), jnp.float32),
        ],
    )(_epilogue_body)
    return f(ainv, *x1planes, *x2planes)


def kernel(input1, input2, eps, iters):
    b, n, _ = input1.shape
    x2t = jnp.transpose(input2, (0, 2, 1))
    x1planes = [input1[:, :, k] for k in range(3)]
    x2planes = [input2[:, :, k] for k in range(3)]
    eps_a = jnp.asarray(eps, jnp.float32).reshape(1)
    it_a = jnp.asarray(iters, jnp.int32).reshape(1)
    ainv3 = pl.pallas_call(
        _auction_body,
        grid=(b,),
        in_specs=[
            pl.BlockSpec(memory_space=pltpu.SMEM),
            pl.BlockSpec(memory_space=pltpu.SMEM),
            pl.BlockSpec((1, n, 3), lambda i: (i, 0, 0)),
            pl.BlockSpec((1, 3, n), lambda i: (i, 0, 0)),
        ],
        out_specs=pl.BlockSpec((1, 1, n), lambda i: (i, 0, 0)),
        out_shape=jax.ShapeDtypeStruct((b, 1, n), jnp.int32),
        scratch_shapes=[pltpu.VMEM((n, n), jnp.float32)],
    )(eps_a, it_a, input1, x2t)
    ass, dist = _epilogue(ainv3[:, 0, :], x1planes, x2planes)
    return dist, ass


# speedup vs baseline: 1.9026x; 1.9026x over previous
import functools

import jax
import jax.numpy as jnp
from jax import lax
from jax.experimental import pallas as pl
from jax.experimental.pallas import tpu as pltpu
from jax.experimental.pallas import tpu_sc as plsc

_N = 1024
_L = 16
_NC = 2
_NS = 16


def _sc_auction_body(eps_hbm, it_hbm, x1a, x1b, x1c, x2a, x2b, x2c,
                     ass_hbm, dist_hbm,
                     par_v, x1p0, x1p1, x1p2, x2p0, x2p1, x2p2,
                     price_v, ainv_v, unass_v, maxinc_v, winner_v,
                     ass_v, dist_v):
    n = _N
    nb = ass_hbm.shape[0]
    wid = lax.axis_index("s") * _NC + lax.axis_index("c")
    lane = lax.broadcasted_iota(jnp.int32, (_L,), 0)
    ninf = jnp.float32(-jnp.inf)
    nblk = n // _L

    def one_batch(b):
        pltpu.sync_copy(eps_hbm, par_v)
        eps = par_v[pl.ds(0, _L)][0]
        pltpu.sync_copy(it_hbm, par_v)
        iters = par_v[pl.ds(0, _L)][0].astype(jnp.int32)
        pltpu.sync_copy(x1a.at[b], x1p0)
        pltpu.sync_copy(x1b.at[b], x1p1)
        pltpu.sync_copy(x1c.at[b], x1p2)
        pltpu.sync_copy(x2a.at[b], x2p0)
        pltpu.sync_copy(x2b.at[b], x2p1)
        pltpu.sync_copy(x2c.at[b], x2p2)

        def init_blk(ib, c):
            sl = pl.ds(ib * _L, _L)
            price_v[sl] = jnp.zeros((_L,), jnp.float32)
            ainv_v[sl] = jnp.full((_L,), -1, jnp.int32)
            unass_v[sl] = jnp.full((_L,), 1, jnp.int32)
            maxinc_v[sl] = jnp.full((_L,), ninf, jnp.float32)
            winner_v[sl] = jnp.full((_L,), n, jnp.int32)
            ass_v[sl] = jnp.full((_L,), -1, jnp.int32)
            dist_v[sl] = jnp.zeros((_L,), jnp.float32)
            return c

        lax.fori_loop(0, nblk, init_blk, 0)

        def scan_row(i):
            isp = jnp.full((_L,), i, jnp.int32)
            x1s0 = plsc.load_gather(x1p0, [isp])
            x1s1 = plsc.load_gather(x1p1, [isp])
            x1s2 = plsc.load_gather(x1p2, [isp])

            def chunk(ch, carry):
                m1, m2, i1 = carry
                sl = pl.ds(ch * _L, _L)
                d0 = x1s0 - x2p0[sl]
                d1 = x1s1 - x2p1[sl]
                d2 = x1s2 - x2p2[sl]
                w = ((d0 * d0 + d1 * d1) + d2 * d2) + price_v[sl]
                idx = ch * _L + lane
                lt = w < m1
                lt2 = w < m2
                m2n = jnp.where(lt, m1, jnp.where(lt2, w, m2))
                i1n = jnp.where(lt, idx, i1)
                m1n = jnp.where(lt, w, m1)
                return m1n, m2n, i1n

            inf16 = jnp.full((_L,), jnp.inf, jnp.float32)
            m1, m2, i1 = lax.fori_loop(
                0, nblk, chunk, (inf16, inf16, jnp.zeros((_L,), jnp.int32)))
            best = jnp.min(m1)
            isb = m1 == best
            cnt = jnp.sum(jnp.where(isb, 1, 0))
            bidx = jnp.min(jnp.where(isb, i1, n))
            sec0 = jnp.min(jnp.where(isb, m2, m1))
            secs = jnp.where(cnt > 1, best, sec0)
            binc = (secs - best) + eps

            bsp = jnp.full((_L,), bidx, jnp.int32)
            cur = plsc.load_gather(maxinc_v, [bsp])
            curw = plsc.load_gather(winner_v, [bsp])
            bincs = jnp.full((_L,), binc)
            better = bincs > cur
            tie = bincs == cur
            neww = jnp.where(better, isp,
                             jnp.where(tie, jnp.minimum(curw, isp), curw))
            plsc.store_scatter(maxinc_v, [bsp], jnp.maximum(cur, bincs),
                               mask=lane == 0)
            plsc.store_scatter(winner_v, [bsp], neww, mask=lane == 0)

        def phase_a(ib, c):
            flags = unass_v[pl.ds(ib * _L, _L)]

            @pl.when(jnp.max(flags) > 0)
            def _():
                for r in range(_L):
                    @pl.when(flags[r] > 0)
                    def _():
                        scan_row(ib * _L + r)

            return c

        def phase_b(ch, c):
            sl = pl.ds(ch * _L, _L)
            mi = maxinc_v[sl]
            hb = mi > ninf
            win = winner_v[sl]
            prev = ainv_v[sl]
            price_v[sl] = jnp.where(hb, price_v[sl] + mi, price_v[sl])
            ainv_v[sl] = jnp.where(hb, win, prev)
            mprev = hb & (prev >= 0)
            plsc.store_scatter(unass_v, [jnp.maximum(prev, 0)],
                               jnp.full((_L,), 1, jnp.int32), mask=mprev)
            plsc.store_scatter(unass_v, [jnp.where(hb, win, 0)],
                               jnp.zeros((_L,), jnp.int32), mask=hb)
            maxinc_v[sl] = jnp.full((_L,), ninf, jnp.float32)
            winner_v[sl] = jnp.full((_L,), n, jnp.int32)
            return c

        def round_body(t, c):
            lax.fori_loop(0, nblk, phase_a, 0)
            lax.fori_loop(0, nblk, phase_b, 0)
            return c

        lax.fori_loop(0, iters, round_body, 0)

        def epi(ch, c):
            sl = pl.ds(ch * _L, _L)
            jids = ch * _L + lane
            owners = ainv_v[sl]
            m = owners >= 0
            oc = jnp.maximum(owners, 0)
            d0 = plsc.load_gather(x1p0, [oc]) - x2p0[sl]
            d1 = plsc.load_gather(x1p1, [oc]) - x2p1[sl]
            d2 = plsc.load_gather(x1p2, [oc]) - x2p2[sl]
            dd = (d0 * d0 + d1 * d1) + d2 * d2
            plsc.store_scatter(dist_v, [oc], dd, mask=m)
            plsc.store_scatter(ass_v, [oc], jids, mask=m)
            return c

        lax.fori_loop(0, nblk, epi, 0)
        pltpu.sync_copy(ass_v, ass_hbm.at[b])
        pltpu.sync_copy(dist_v, dist_hbm.at[b])

    @pl.when(wid < nb)
    def _():
        one_batch(wid)


def kernel(input1, input2, eps, iters):
    b, n, _ = input1.shape
    x1planes = [input1[:, :, k] for k in range(3)]
    x2planes = [input2[:, :, k] for k in range(3)]
    eps_a = jnp.full((_L,), eps, jnp.float32)
    it_a = jnp.full((_L,), iters, jnp.float32)
    mesh = plsc.VectorSubcoreMesh(
        core_axis_name="c", subcore_axis_name="s",
        num_cores=_NC, num_subcores=_NS)
    f = functools.partial(
        pl.kernel,
        out_type=[
            jax.ShapeDtypeStruct((b, _N), jnp.int32),
            jax.ShapeDtypeStruct((b, _N), jnp.float32),
        ],
        mesh=mesh,
        compiler_params=pltpu.CompilerParams(
            use_tc_tiling_on_sc=False, needs_layout_passes=False),
        scratch_types=(
            [pltpu.VMEM((_L,), jnp.float32)]
            + [pltpu.VMEM((_N,), jnp.float32)] * 6
            + [pltpu.VMEM((_N,), jnp.float32),
               pltpu.VMEM((_N,), jnp.int32),
               pltpu.VMEM((_N,), jnp.int32),
               pltpu.VMEM((_N,), jnp.float32),
               pltpu.VMEM((_N,), jnp.int32),
               pltpu.VMEM((_N,), jnp.int32),
               pltpu.VMEM((_N,), jnp.float32)]
        ),
    )(_sc_auction_body)
    ass, dist = f(eps_a, it_a, *x1planes, *x2planes)
    return dist, ass
